# Initial kernel scaffold; baseline (speedup 1.0000x reference)
import functools

import jax
import jax.numpy as jnp
from jax import lax
from jax.experimental import pallas as pl
from jax.experimental.pallas import tpu as pltpu
from jax.experimental.pallas import tpu_sc as plsc

NN = 10000
EE = 320000
RR = 4
DH = 128
NP = 10240
NC, NS = 2, 16
NW = NC * NS
EW = 10240
EP = NW * EW
KC = 128
NCH = EW // KC
BN_ = 256

_mesh = plsc.VectorSubcoreMesh(
    core_axis_name="c", subcore_axis_name="s", num_cores=NC, num_subcores=NS)


@functools.partial(
    pl.kernel,
    out_type=jax.ShapeDtypeStruct((NC, RR * NP, 16), jnp.float32),
    mesh=_mesh,
    scratch_types=[
        pltpu.VMEM((KC,), jnp.int32),
        pltpu.VMEM((KC, 16), jnp.float32),
        pltpu.VMEM((KC, 16), jnp.float32),
        pltpu.VMEM_SHARED((RR * NP, 16), jnp.float32),
        pltpu.SemaphoreType.DMA,
    ],
)
def _count_kernel(dstg_hbm, out_hbm, idx_v, ones_v, tmp_v, hist_s, sem):
    c = lax.axis_index("c")
    s = lax.axis_index("s")
    w = s * NC + c
    zrows = (RR * NP) // NS

    def fill(i, _):
        ones_v[i, :] = jnp.ones((16,), jnp.float32)
        tmp_v[i, :] = jnp.zeros((16,), jnp.float32)
        return ()
    lax.fori_loop(0, KC, fill, ())

    def zero(i, _):
        pltpu.sync_copy(tmp_v, hist_s.at[pl.ds(s * zrows + i * KC, KC)])
        return ()
    lax.fori_loop(0, zrows // KC, zero, ())
    plsc.subcore_barrier()

    def body(i, _):
        pltpu.sync_copy(dstg_hbm.at[pl.ds(w * EW + i * KC, KC)], idx_v)
        pltpu.sync_copy(ones_v, hist_s.at[idx_v], add=True)
        return ()
    lax.fori_loop(0, NCH, body, ())
    plsc.subcore_barrier()

    def out(i, _):
        pltpu.sync_copy(hist_s.at[pl.ds(s * zrows + i * KC, KC)], tmp_v)
        pltpu.sync_copy(tmp_v, out_hbm.at[c, pl.ds(s * zrows + i * KC, KC)])
        return ()
    lax.fori_loop(0, zrows // KC, out, ())


@functools.partial(
    pl.kernel,
    out_type=jax.ShapeDtypeStruct((NC, NP, DH), jnp.float32),
    mesh=_mesh,
    scratch_types=[
        pltpu.VMEM((KC,), jnp.int32),
        pltpu.VMEM((KC,), jnp.int32),
        pltpu.VMEM((KC,), jnp.int32),
        pltpu.VMEM((KC, DH), jnp.float32),
        pltpu.VMEM((KC, 2 * DH), jnp.float32),
        pltpu.VMEM((KC, DH), jnp.float32),
        pltpu.VMEM_SHARED((NP, DH), jnp.float32),
        pltpu.SemaphoreType.DMA,
        pltpu.SemaphoreType.DMA,
    ],
)
def _edge_kernel(xr_hbm, gb_hbm, srcg_hbm, dstg_hbm, dst_hbm, out_hbm,
                 sidx, gidx, didx, xr_v, gb_v, msg_v, acc_s, sem1, sem2):
    c = lax.axis_index("c")
    s = lax.axis_index("s")
    w = s * NC + c
    zrows = NP // NS

    def zfill(i, _):
        for g in range(DH // 16):
            msg_v[i, pl.ds(g * 16, 16)] = jnp.zeros((16,), jnp.float32)
        return ()
    lax.fori_loop(0, KC, zfill, ())

    def zero(i, _):
        pltpu.sync_copy(msg_v, acc_s.at[pl.ds(s * zrows + i * KC, KC)])
        return ()
    lax.fori_loop(0, zrows // KC, zero, ())
    plsc.subcore_barrier()

    def body(i, _):
        off = w * EW + i * KC
        pltpu.sync_copy(srcg_hbm.at[pl.ds(off, KC)], sidx)
        pltpu.sync_copy(dstg_hbm.at[pl.ds(off, KC)], gidx)
        pltpu.sync_copy(dst_hbm.at[pl.ds(off, KC)], didx)
        cp1 = pltpu.async_copy(xr_hbm.at[sidx], xr_v, sem1)
        cp2 = pltpu.async_copy(gb_hbm.at[gidx], gb_v, sem2)
        cp1.wait()
        cp2.wait()

        def edge(e, _):
            for g in range(DH // 16):
                xr = xr_v[e, pl.ds(g * 16, 16)]
                beta = gb_v[e, pl.ds(g * 16, 16)]
                gamma = gb_v[e, pl.ds(DH + g * 16, 16)]
                msg_v[e, pl.ds(g * 16, 16)] = jnp.maximum(
                    gamma * xr + beta, 0.0)
            return ()
        lax.fori_loop(0, KC, edge, ())
        pltpu.sync_copy(msg_v, acc_s.at[didx], add=True)
        return ()
    lax.fori_loop(0, NCH, body, ())
    plsc.subcore_barrier()

    def out(i, _):
        pltpu.sync_copy(acc_s.at[pl.ds(s * zrows + i * KC, KC)], msg_v)
        pltpu.sync_copy(msg_v, out_hbm.at[c, pl.ds(s * zrows + i * KC, KC)])
        return ()
    lax.fori_loop(0, zrows // KC, out, ())


def _dense_body(x_ref, w_ref, cnt_ref, xr_ref, gb_ref, skip_ref):
    y = jnp.dot(x_ref[...], w_ref[...], preferred_element_type=jnp.float32)
    cnt = jnp.max(cnt_ref[...], axis=-1)
    inv = 1.0 / jnp.maximum(cnt[0] + cnt[1], 1.0)
    for r in range(RR):
        xr_ref[r] = y[:, r * DH:(r + 1) * DH]
        gb_ref[r] = (y[:, RR * DH + r * 2 * DH: RR * DH + (r + 1) * 2 * DH]
                     * inv[r][:, None])
    base = 3 * RR * DH
    s_lin = y[:, base:base + DH]
    bs = y[:, base + DH:base + 2 * DH]
    gs = y[:, base + 2 * DH:base + 3 * DH]
    skip_ref[...] = jnp.maximum(gs * s_lin + bs, 0.0)


_dense = pl.pallas_call(
    _dense_body,
    grid=(NP // BN_,),
    in_specs=[
        pl.BlockSpec((BN_, DH), lambda i: (i, 0)),
        pl.BlockSpec((DH, 3 * RR * DH + 3 * DH), lambda i: (0, 0)),
        pl.BlockSpec((NC, RR, BN_, 16), lambda i: (0, 0, i, 0)),
    ],
    out_specs=[
        pl.BlockSpec((RR, BN_, DH), lambda i: (0, i, 0)),
        pl.BlockSpec((RR, BN_, 2 * DH), lambda i: (0, i, 0)),
        pl.BlockSpec((BN_, DH), lambda i: (i, 0)),
    ],
    out_shape=[
        jax.ShapeDtypeStruct((RR, NP, DH), jnp.float32),
        jax.ShapeDtypeStruct((RR, NP, 2 * DH), jnp.float32),
        jax.ShapeDtypeStruct((NP, DH), jnp.float32),
    ],
)


def _bn_body(skip_ref, p0_ref, p1_ref, g_ref, b_ref, o_ref):
    h = skip_ref[...] + p0_ref[...] + p1_ref[...]
    mu = jnp.mean(h, axis=0, keepdims=True)
    var = jnp.mean(h * h, axis=0, keepdims=True) - mu * mu
    hn = g_ref[...] * (h - mu) * lax.rsqrt(var + 1e-5) + b_ref[...]
    o_ref[...] = jnp.maximum(hn, 0.0)


_bn = pl.pallas_call(
    _bn_body,
    out_shape=jax.ShapeDtypeStruct((NN, DH), jnp.float32),
)


def _add_body(skip_ref, p0_ref, p1_ref, o_ref):
    o_ref[...] = skip_ref[...] + p0_ref[...] + p1_ref[...]


_add = pl.pallas_call(
    _add_body,
    out_shape=jax.ShapeDtypeStruct((NN, DH), jnp.float32),
)


def _wcat(W, F, Ws, Fs):
    return jnp.concatenate(
        [W[r] for r in range(RR)] + [F[r] for r in range(RR)] + [Ws, Fs],
        axis=1)


def kernel(x, edge_index, edge_type, W0, F0, Ws0, Fs0, bn_g, bn_b,
           W1, F1, Ws1, Fs1):
    src = edge_index[0].astype(jnp.int32)
    dst = edge_index[1].astype(jnp.int32)
    et = edge_type.astype(jnp.int32)
    pad = jnp.full((EP - EE,), NP - 1, jnp.int32)
    src_p = jnp.concatenate([src, pad])
    dst_p = jnp.concatenate([dst, pad])
    et_p = jnp.concatenate([et, jnp.zeros((EP - EE,), jnp.int32)])
    src_g = et_p * NP + src_p
    dst_g = et_p * NP + dst_p

    cnt = _count_kernel(dst_g).reshape(NC, RR, NP, 16)

    xpad = jnp.pad(x, ((0, NP - NN), (0, 0)))
    xr0, gb0, skip0 = _dense(xpad, _wcat(W0, F0, Ws0, Fs0), cnt)
    part0 = _edge_kernel(xr0.reshape(RR * NP, DH),
                         gb0.reshape(RR * NP, 2 * DH), src_g, dst_g, dst_p)
    h = _bn(skip0[:NN], part0[0, :NN], part0[1, :NN],
            bn_g.reshape(1, DH), bn_b.reshape(1, DH))

    hpad = jnp.pad(h, ((0, NP - NN), (0, 0)))
    xr1, gb1, skip1 = _dense(hpad, _wcat(W1, F1, Ws1, Fs1), cnt)
    part1 = _edge_kernel(xr1.reshape(RR * NP, DH),
                         gb1.reshape(RR * NP, 2 * DH), src_g, dst_g, dst_p)
    return _add(skip1[:NN], part1[0, :NN], part1[1, :NN])


# speedup vs baseline: 3.1845x; 3.1845x over previous
import functools

import jax
import jax.numpy as jnp
from jax import lax
from jax.experimental import pallas as pl
from jax.experimental.pallas import tpu as pltpu
from jax.experimental.pallas import tpu_sc as plsc

NN = 10000
EE = 320000
RR = 4
DH = 128
NP = 10240
NC, NS = 2, 16
NW = NC * NS
EW = 10240
EP = NW * EW
KC = 128
NCH = EW // KC
BN_ = 256
MSK = (1 << 14) - 1

_mesh = plsc.VectorSubcoreMesh(
    core_axis_name="c", subcore_axis_name="s", num_cores=NC, num_subcores=NS)


NH = NP // NC
ACR = NH + KC
ECS = EP // KC // NS


@functools.partial(
    pl.kernel,
    out_type=jax.ShapeDtypeStruct((NC, NH, DH), jnp.float32),
    mesh=_mesh,
    scratch_types=[
        pltpu.VMEM((KC,), jnp.int32),
        pltpu.VMEM((KC,), jnp.int32),
        pltpu.VMEM((KC,), jnp.int32),
        pltpu.VMEM((KC,), jnp.int32),
        pltpu.VMEM((KC, DH), jnp.float32),
        pltpu.VMEM((KC, 2 * DH), jnp.float32),
        pltpu.VMEM((KC, DH), jnp.float32),
        pltpu.VMEM_SHARED((ACR, DH), jnp.float32),
        pltpu.SemaphoreType.DMA,
        pltpu.SemaphoreType.DMA,
    ],
)
def _edge_kernel(xr_hbm, gb_hbm, code_hbm, out_hbm,
                 code_v, sidx, gidx, didx, xr_v, gb_v, msg_v, acc_s,
                 sem1, sem2):
    c = lax.axis_index("c")
    s = lax.axis_index("s")
    vzero = jnp.zeros((16,), jnp.float32)

    def zfill(i, _):
        for g in range(DH // 16):
            msg_v[i, pl.ds(g * 16, 16)] = vzero
        return ()
    lax.fori_loop(0, KC, zfill, ())

    def zero(k, _):
        i = s + k * NS

        @pl.when(i < ACR // KC)
        def _():
            pltpu.sync_copy(msg_v, acc_s.at[pl.ds(i * KC, KC)])
        return ()
    lax.fori_loop(0, (ACR // KC + NS - 1) // NS, zero, ())
    plsc.subcore_barrier()

    def body(i, _):
        pltpu.sync_copy(code_hbm.at[pl.ds((s * ECS + i) * KC, KC)], code_v)

        def unpack(j, _):
            c16 = code_v[pl.ds(j * 16, 16)]
            t16 = lax.shift_right_logical(c16, 28)
            s16 = lax.shift_right_logical(c16, 14) & MSK
            d16 = c16 & MSK
            sidx[pl.ds(j * 16, 16)] = t16 * NP + s16
            gidx[pl.ds(j * 16, 16)] = t16 * NP + d16
            loc = d16 - c * NH
            didx[pl.ds(j * 16, 16)] = jnp.where(
                (loc >= 0) & (loc < NH), loc, NH)
            return ()
        lax.fori_loop(0, KC // 16, unpack, ())

        cp1 = pltpu.async_copy(xr_hbm.at[sidx], xr_v, sem1)
        cp2 = pltpu.async_copy(gb_hbm.at[gidx], gb_v, sem2)
        cp1.wait()
        cp2.wait()

        def edge(e, _):
            for g in range(DH // 16):
                xr = xr_v[e, pl.ds(g * 16, 16)]
                beta = gb_v[e, pl.ds(g * 16, 16)]
                gamma = gb_v[e, pl.ds(DH + g * 16, 16)]
                msg_v[e, pl.ds(g * 16, 16)] = jnp.maximum(
                    gamma * xr + beta, 0.0)
            return ()
        lax.fori_loop(0, KC, edge, ())
        pltpu.sync_copy(msg_v, acc_s.at[didx], add=True)
        return ()
    lax.fori_loop(0, ECS, body, ())
    plsc.subcore_barrier()

    def out(k, _):
        i = s + k * NS

        @pl.when(i < NH // KC)
        def _():
            pltpu.sync_copy(acc_s.at[pl.ds(i * KC, KC)], msg_v)
            pltpu.sync_copy(msg_v, out_hbm.at[c, pl.ds(i * KC, KC)])
        return ()
    lax.fori_loop(0, (NH // KC + NS - 1) // NS, out, ())


def _dense_body(x_ref, w_ref, cnt_ref, xr_ref, gb_ref, skip_ref):
    y = jnp.dot(x_ref[...], w_ref[...], preferred_element_type=jnp.float32)
    cnt = cnt_ref[...]
    inv = 1.0 / jnp.maximum(cnt, 1.0)
    for r in range(RR):
        xr_ref[r] = y[:, r * DH:(r + 1) * DH]
        gb_ref[r] = (y[:, RR * DH + r * 2 * DH: RR * DH + (r + 1) * 2 * DH]
                     * inv[:, r][:, None])
    base = 3 * RR * DH
    s_lin = y[:, base:base + DH]
    bs = y[:, base + DH:base + 2 * DH]
    gs = y[:, base + 2 * DH:base + 3 * DH]
    skip_ref[...] = jnp.maximum(gs * s_lin + bs, 0.0)


_dense = pl.pallas_call(
    _dense_body,
    grid=(NP // BN_,),
    in_specs=[
        pl.BlockSpec((BN_, DH), lambda i: (i, 0)),
        pl.BlockSpec((DH, 3 * RR * DH + 3 * DH), lambda i: (0, 0)),
        pl.BlockSpec((BN_, 16), lambda i: (i, 0)),
    ],
    out_specs=[
        pl.BlockSpec((RR, BN_, DH), lambda i: (0, i, 0)),
        pl.BlockSpec((RR, BN_, 2 * DH), lambda i: (0, i, 0)),
        pl.BlockSpec((BN_, DH), lambda i: (i, 0)),
    ],
    out_shape=[
        jax.ShapeDtypeStruct((RR, NP, DH), jnp.float32),
        jax.ShapeDtypeStruct((RR, NP, 2 * DH), jnp.float32),
        jax.ShapeDtypeStruct((NP, DH), jnp.float32),
    ],
)


def _bn_body(skip_ref, p_ref, g_ref, b_ref, o_ref):
    h = skip_ref[...] + p_ref[...]
    mu = jnp.mean(h, axis=0, keepdims=True)
    var = jnp.mean(h * h, axis=0, keepdims=True) - mu * mu
    hn = g_ref[...] * (h - mu) * lax.rsqrt(var + 1e-5) + b_ref[...]
    o_ref[...] = jnp.maximum(hn, 0.0)


_bn = pl.pallas_call(
    _bn_body,
    out_shape=jax.ShapeDtypeStruct((NN, DH), jnp.float32),
)


def _add_body(skip_ref, p_ref, o_ref):
    o_ref[...] = skip_ref[...] + p_ref[...]


_add = pl.pallas_call(
    _add_body,
    out_shape=jax.ShapeDtypeStruct((NN, DH), jnp.float32),
)


def _wcat(W, F, Ws, Fs):
    return jnp.concatenate(
        [W[r] for r in range(RR)] + [F[r] for r in range(RR)] + [Ws, Fs],
        axis=1)


def kernel(x, edge_index, edge_type, W0, F0, Ws0, Fs0, bn_g, bn_b,
           W1, F1, Ws1, Fs1):
    src = edge_index[0].astype(jnp.int32)
    dst = edge_index[1].astype(jnp.int32)
    et = edge_type.astype(jnp.int32)
    code = (et << 28) | (src << 14) | dst
    code = jnp.concatenate(
        [code, jnp.full((EP - EE,), ((NP - 1) << 14) | (NP - 1), jnp.int32)])

    lane = jnp.arange(DH, dtype=jnp.int32)
    row_rel = (jnp.arange(RR * NP, dtype=jnp.int32) // NP)
    bconst = (lane[None, :] == row_rel[:, None]).astype(jnp.float32)
    gb_const = jnp.concatenate([bconst, jnp.zeros_like(bconst)], axis=1)
    xr_const = jnp.zeros((RR * NP, DH), jnp.float32)
    cnt_full = _edge_kernel(xr_const, gb_const, code).reshape(NP, DH)
    cnt = cnt_full[:, :16]

    def edge_phase(xr, gb):
        return _edge_kernel(xr.reshape(RR * NP, DH),
                            gb.reshape(RR * NP, 2 * DH), code).reshape(NP, DH)

    xpad = jnp.pad(x, ((0, NP - NN), (0, 0)))
    xr0, gb0, skip0 = _dense(xpad, _wcat(W0, F0, Ws0, Fs0), cnt)
    part0 = edge_phase(xr0, gb0)
    h = _bn(skip0[:NN], part0[:NN],
            bn_g.reshape(1, DH), bn_b.reshape(1, DH))

    hpad = jnp.pad(h, ((0, NP - NN), (0, 0)))
    xr1, gb1, skip1 = _dense(hpad, _wcat(W1, F1, Ws1, Fs1), cnt)
    part1 = edge_phase(xr1, gb1)
    return _add(skip1[:NN], part1[:NN])


# speedup vs baseline: 4.5269x; 1.4216x over previous
import functools

import jax
import jax.numpy as jnp
from jax import lax
from jax.experimental import pallas as pl
from jax.experimental.pallas import tpu as pltpu
from jax.experimental.pallas import tpu_sc as plsc

NN = 10000
EE = 320000
RR = 4
DH = 128
NP = 10240
NC, NS = 2, 16
NW = NC * NS
EW = 10240
EP = NW * EW
KC = 128
NCH = EW // KC
BN_ = 256
MSK = (1 << 14) - 1

_mesh = plsc.VectorSubcoreMesh(
    core_axis_name="c", subcore_axis_name="s", num_cores=NC, num_subcores=NS)


NH = NP // NC
ACR = NH + KC
ECS = EP // KC // NS


@functools.partial(
    pl.kernel,
    out_type=jax.ShapeDtypeStruct((NC, NH, DH), jnp.float32),
    mesh=_mesh,
    scratch_types=[
        pltpu.VMEM((KC,), jnp.int32),
        pltpu.VMEM((KC,), jnp.int32),
        pltpu.VMEM((KC,), jnp.int32),
        pltpu.VMEM((KC,), jnp.int32),
        pltpu.VMEM((KC, DH), jnp.float32),
        pltpu.VMEM((KC, 2 * DH), jnp.float32),
        pltpu.VMEM((KC, DH), jnp.float32),
        pltpu.VMEM_SHARED((ACR, DH), jnp.float32),
        pltpu.SemaphoreType.DMA,
        pltpu.SemaphoreType.DMA,
    ],
)
def _edge_kernel(xr_hbm, gb_hbm, code_hbm, out_hbm,
                 code_v, sidx, gidx, didx, xr_v, gb_v, msg_v, acc_s,
                 sem1, sem2):
    c = lax.axis_index("c")
    s = lax.axis_index("s")
    vzero = jnp.zeros((16,), jnp.float32)

    def zfill(i, _):
        for g in range(DH // 16):
            msg_v[i, pl.ds(g * 16, 16)] = vzero
        return ()
    lax.fori_loop(0, KC, zfill, ())

    def zero(k, _):
        i = s + k * NS

        @pl.when(i < ACR // KC)
        def _():
            pltpu.sync_copy(msg_v, acc_s.at[pl.ds(i * KC, KC)])
        return ()
    lax.fori_loop(0, (ACR // KC + NS - 1) // NS, zero, ())
    plsc.subcore_barrier()

    def body(i, _):
        pltpu.sync_copy(code_hbm.at[pl.ds((s * ECS + i) * KC, KC)], code_v)

        def unpack(j, _):
            c16 = code_v[pl.ds(j * 16, 16)]
            t16 = lax.shift_right_logical(c16, 28)
            s16 = lax.shift_right_logical(c16, 14) & MSK
            d16 = c16 & MSK
            sidx[pl.ds(j * 16, 16)] = t16 * NP + s16
            gidx[pl.ds(j * 16, 16)] = t16 * NP + d16
            loc = d16 - c * NH
            didx[pl.ds(j * 16, 16)] = jnp.where(
                (loc >= 0) & (loc < NH), loc, NH)
            return ()
        lax.fori_loop(0, KC // 16, unpack, ())

        cp1 = pltpu.async_copy(xr_hbm.at[sidx], xr_v, sem1)
        cp2 = pltpu.async_copy(gb_hbm.at[gidx], gb_v, sem2)
        cp1.wait()
        cp2.wait()

        def edge(e, _):
            for g in range(DH // 16):
                xr = xr_v[e, pl.ds(g * 16, 16)]
                beta = gb_v[e, pl.ds(g * 16, 16)]
                gamma = gb_v[e, pl.ds(DH + g * 16, 16)]
                msg_v[e, pl.ds(g * 16, 16)] = jnp.maximum(
                    gamma * xr + beta, 0.0)
            return ()
        lax.fori_loop(0, KC, edge, ())
        pltpu.sync_copy(msg_v, acc_s.at[didx], add=True)
        return ()
    lax.fori_loop(0, ECS, body, ())
    plsc.subcore_barrier()

    def out(k, _):
        i = s + k * NS

        @pl.when(i < NH // KC)
        def _():
            pltpu.sync_copy(acc_s.at[pl.ds(i * KC, KC)], msg_v)
            pltpu.sync_copy(msg_v, out_hbm.at[c, pl.ds(i * KC, KC)])
        return ()
    lax.fori_loop(0, (NH // KC + NS - 1) // NS, out, ())


@functools.partial(
    pl.kernel,
    out_type=jax.ShapeDtypeStruct((NC, NH, DH), jnp.float32),
    mesh=_mesh,
    scratch_types=[
        pltpu.VMEM((KC,), jnp.int32),
        pltpu.VMEM((KC,), jnp.int32),
        pltpu.VMEM((KC, DH), jnp.float32),
        pltpu.VMEM_SHARED((ACR, DH), jnp.float32),
        pltpu.SemaphoreType.DMA,
    ],
)
def _count_kernel(code_hbm, out_hbm, code_v, didx, msg_v, acc_s, sem):
    c = lax.axis_index("c")
    s = lax.axis_index("s")
    vzero = jnp.zeros((16,), jnp.float32)
    lanes = lax.broadcasted_iota(jnp.int32, (16,), 0)

    def zfill(i, _):
        for g in range(DH // 16):
            msg_v[i, pl.ds(g * 16, 16)] = vzero
        return ()
    lax.fori_loop(0, KC, zfill, ())

    def zero(k, _):
        i = s + k * NS

        @pl.when(i < ACR // KC)
        def _():
            pltpu.sync_copy(msg_v, acc_s.at[pl.ds(i * KC, KC)])
        return ()
    lax.fori_loop(0, (ACR // KC + NS - 1) // NS, zero, ())
    plsc.subcore_barrier()

    def body(i, _):
        pltpu.sync_copy(code_hbm.at[pl.ds((s * ECS + i) * KC, KC)], code_v)

        def setrows(j, _):
            c16 = code_v[pl.ds(j * 16, 16)]
            t16 = lax.shift_right_logical(c16, 28)
            d16 = c16 & MSK
            loc = d16 - c * NH
            didx[pl.ds(j * 16, 16)] = jnp.where(
                (loc >= 0) & (loc < NH), loc, NH)
            for k in range(16):
                msg_v[j * 16 + k, pl.ds(0, 16)] = jnp.where(
                    lanes == t16[k], 1.0, 0.0).astype(jnp.float32)
            return ()
        lax.fori_loop(0, KC // 16, setrows, ())
        pltpu.sync_copy(msg_v, acc_s.at[didx], add=True)
        return ()
    lax.fori_loop(0, ECS, body, ())
    plsc.subcore_barrier()

    def out(k, _):
        i = s + k * NS

        @pl.when(i < NH // KC)
        def _():
            pltpu.sync_copy(acc_s.at[pl.ds(i * KC, KC)], msg_v)
            pltpu.sync_copy(msg_v, out_hbm.at[c, pl.ds(i * KC, KC)])
        return ()
    lax.fori_loop(0, (NH // KC + NS - 1) // NS, out, ())


def _dense_body(x_ref, w_ref, cnt_ref, xr_ref, gb_ref, skip_ref):
    y = jnp.dot(x_ref[...], w_ref[...], preferred_element_type=jnp.float32)
    cnt = cnt_ref[...]
    inv = 1.0 / jnp.maximum(cnt, 1.0)
    for r in range(RR):
        xr_ref[r] = y[:, r * DH:(r + 1) * DH]
        gb_ref[r] = (y[:, RR * DH + r * 2 * DH: RR * DH + (r + 1) * 2 * DH]
                     * inv[:, r][:, None])
    base = 3 * RR * DH
    s_lin = y[:, base:base + DH]
    bs = y[:, base + DH:base + 2 * DH]
    gs = y[:, base + 2 * DH:base + 3 * DH]
    skip_ref[...] = jnp.maximum(gs * s_lin + bs, 0.0)


_dense = pl.pallas_call(
    _dense_body,
    grid=(NP // BN_,),
    in_specs=[
        pl.BlockSpec((BN_, DH), lambda i: (i, 0)),
        pl.BlockSpec((DH, 3 * RR * DH + 3 * DH), lambda i: (0, 0)),
        pl.BlockSpec((BN_, 16), lambda i: (i, 0)),
    ],
    out_specs=[
        pl.BlockSpec((RR, BN_, DH), lambda i: (0, i, 0)),
        pl.BlockSpec((RR, BN_, 2 * DH), lambda i: (0, i, 0)),
        pl.BlockSpec((BN_, DH), lambda i: (i, 0)),
    ],
    out_shape=[
        jax.ShapeDtypeStruct((RR, NP, DH), jnp.float32),
        jax.ShapeDtypeStruct((RR, NP, 2 * DH), jnp.float32),
        jax.ShapeDtypeStruct((NP, DH), jnp.float32),
    ],
)


def _bn_body(skip_ref, p_ref, g_ref, b_ref, o_ref):
    h = skip_ref[...] + p_ref[...]
    mu = jnp.mean(h, axis=0, keepdims=True)
    var = jnp.mean(h * h, axis=0, keepdims=True) - mu * mu
    hn = g_ref[...] * (h - mu) * lax.rsqrt(var + 1e-5) + b_ref[...]
    o_ref[...] = jnp.maximum(hn, 0.0)


_bn = pl.pallas_call(
    _bn_body,
    out_shape=jax.ShapeDtypeStruct((NN, DH), jnp.float32),
)


def _add_body(skip_ref, p_ref, o_ref):
    o_ref[...] = skip_ref[...] + p_ref[...]


_add = pl.pallas_call(
    _add_body,
    out_shape=jax.ShapeDtypeStruct((NN, DH), jnp.float32),
)


def _wcat(W, F, Ws, Fs):
    return jnp.concatenate(
        [W[r] for r in range(RR)] + [F[r] for r in range(RR)] + [Ws, Fs],
        axis=1)


def kernel(x, edge_index, edge_type, W0, F0, Ws0, Fs0, bn_g, bn_b,
           W1, F1, Ws1, Fs1):
    src = edge_index[0].astype(jnp.int32)
    dst = edge_index[1].astype(jnp.int32)
    et = edge_type.astype(jnp.int32)
    code = (et << 28) | (src << 14) | dst
    code = jnp.concatenate(
        [code, jnp.full((EP - EE,), ((NP - 1) << 14) | (NP - 1), jnp.int32)])

    cnt_full = _count_kernel(code).reshape(NP, DH)
    cnt = cnt_full[:, :16]

    def edge_phase(xr, gb):
        return _edge_kernel(xr.reshape(RR * NP, DH),
                            gb.reshape(RR * NP, 2 * DH), code).reshape(NP, DH)

    xpad = jnp.pad(x, ((0, NP - NN), (0, 0)))
    xr0, gb0, skip0 = _dense(xpad, _wcat(W0, F0, Ws0, Fs0), cnt)
    part0 = edge_phase(xr0, gb0)
    h = _bn(skip0[:NN], part0[:NN],
            bn_g.reshape(1, DH), bn_b.reshape(1, DH))

    hpad = jnp.pad(h, ((0, NP - NN), (0, 0)))
    xr1, gb1, skip1 = _dense(hpad, _wcat(W1, F1, Ws1, Fs1), cnt)
    part1 = edge_phase(xr1, gb1)
    return _add(skip1[:NN], part1[:NN])
